# Initial kernel scaffold; baseline (speedup 1.0000x reference)
import jax
import jax.numpy as jnp
from jax.experimental import pallas as pl


def kernel(t, beta, alpha):
    raise NotImplementedError("write your pallas kernel here")


# speedup vs baseline: 5.8621x; 5.8621x over previous
import functools

import jax
import jax.numpy as jnp
from jax import lax
from jax.experimental import pallas as pl
from jax.experimental.pallas import tpu as pltpu
from jax.experimental.pallas import tpu_sc as plsc

N_STEPS = 1000
BATCH = 16384

_info = plsc.get_sparse_core_info()
_NC, _NS, _L = _info.num_cores, _info.num_subcores, _info.num_lanes
_NW = _NC * _NS
_B_PER_W = BATCH // _NW
_CH = 128
_NCH = _B_PER_W // _CH

_mesh = plsc.VectorSubcoreMesh(core_axis_name="c", subcore_axis_name="s")


@functools.partial(
    pl.kernel,
    mesh=_mesh,
    out_type=(
        jax.ShapeDtypeStruct((BATCH,), jnp.float32),
        jax.ShapeDtypeStruct((BATCH,), jnp.float32),
    ),
    scratch_types=[
        pltpu.VMEM((_NCH, _CH), jnp.int32),
        pltpu.VMEM((_NCH, _CH), jnp.float32),
        pltpu.VMEM((_NCH, _CH), jnp.float32),
        pltpu.SemaphoreType.DMA,
    ],
)
def _noise_lookup(t_hbm, beta_hbm, alpha_hbm, bt_hbm, at_hbm,
                  idx_v, ob_v, oa_v, sem):
    wid = lax.axis_index("s") * _NC + lax.axis_index("c")
    base = wid * _B_PER_W
    for j in range(_NCH):
        pltpu.sync_copy(t_hbm.at[pl.ds(base + j * _CH, _CH)], idx_v.at[j])
    copies = []
    for j in range(_NCH):
        copies.append(pltpu.async_copy(beta_hbm.at[idx_v.at[j]], ob_v.at[j], sem))
        copies.append(pltpu.async_copy(alpha_hbm.at[idx_v.at[j]], oa_v.at[j], sem))
    for c in copies:
        c.wait()
    for j in range(_NCH):
        pltpu.sync_copy(ob_v.at[j], bt_hbm.at[pl.ds(base + j * _CH, _CH)])
        pltpu.sync_copy(oa_v.at[j], at_hbm.at[pl.ds(base + j * _CH, _CH)])


def kernel(t, beta, alpha):
    return _noise_lookup(t, beta, alpha)


# speedup vs baseline: 6.1318x; 1.0460x over previous
import functools

import jax
import jax.numpy as jnp
from jax import lax
from jax.experimental import pallas as pl
from jax.experimental.pallas import tpu as pltpu
from jax.experimental.pallas import tpu_sc as plsc

N_STEPS = 1000
BATCH = 16384

_info = plsc.get_sparse_core_info()
_NC, _NS, _L = _info.num_cores, _info.num_subcores, _info.num_lanes
_NW = _NC * _NS
_CH = 128
_NROWS = BATCH // _CH
_R_PER_W = _NROWS // _NW

_mesh = plsc.VectorSubcoreMesh(core_axis_name="c", subcore_axis_name="s")


@functools.partial(
    pl.kernel,
    mesh=_mesh,
    out_type=(
        jax.ShapeDtypeStruct((_NROWS, _CH), jnp.float32),
        jax.ShapeDtypeStruct((_NROWS, _CH), jnp.float32),
    ),
    scratch_types=[
        pltpu.VMEM((_R_PER_W, _CH), jnp.int32),
        pltpu.VMEM((_R_PER_W, _CH), jnp.float32),
        pltpu.VMEM((_R_PER_W, _CH), jnp.float32),
        pltpu.SemaphoreType.DMA,
        pltpu.SemaphoreType.DMA,
    ],
)
def _noise_lookup(t_hbm, beta_hbm, alpha_hbm, bt_hbm, at_hbm,
                  idx_v, ob_v, oa_v, sem_g, sem_o):
    wid = lax.axis_index("s") * _NC + lax.axis_index("c")
    row0 = wid * _R_PER_W
    pltpu.sync_copy(t_hbm.at[pl.ds(row0, _R_PER_W)], idx_v)
    copies = []
    for j in range(_R_PER_W):
        copies.append(pltpu.async_copy(beta_hbm.at[idx_v.at[j]], ob_v.at[j], sem_g))
        copies.append(pltpu.async_copy(alpha_hbm.at[idx_v.at[j]], oa_v.at[j], sem_g))
    for c in copies:
        c.wait()
    ob_c = pltpu.async_copy(ob_v, bt_hbm.at[pl.ds(row0, _R_PER_W)], sem_o)
    oa_c = pltpu.async_copy(oa_v, at_hbm.at[pl.ds(row0, _R_PER_W)], sem_o)
    ob_c.wait()
    oa_c.wait()


def kernel(t, beta, alpha):
    bt, at = _noise_lookup(t.reshape(_NROWS, _CH), beta, alpha)
    return bt.reshape(BATCH), at.reshape(BATCH)


# speedup vs baseline: 6.4349x; 1.0494x over previous
import functools

import jax
import jax.numpy as jnp
from jax import lax
from jax.experimental import pallas as pl
from jax.experimental.pallas import tpu as pltpu
from jax.experimental.pallas import tpu_sc as plsc

N_STEPS = 1000
BATCH = 16384

_info = plsc.get_sparse_core_info()
_NC, _NS, _L = 1, _info.num_subcores, _info.num_lanes
_NW = _NC * _NS
_CH = 128
_NROWS = BATCH // _CH
_R_PER_W = _NROWS // _NW

_mesh = plsc.VectorSubcoreMesh(core_axis_name="c", subcore_axis_name="s",
                               num_cores=1)


@functools.partial(
    pl.kernel,
    mesh=_mesh,
    out_type=(
        jax.ShapeDtypeStruct((_NROWS, _CH), jnp.float32),
        jax.ShapeDtypeStruct((_NROWS, _CH), jnp.float32),
    ),
    scratch_types=[
        pltpu.VMEM((_R_PER_W, _CH), jnp.int32),
        pltpu.VMEM((_R_PER_W, _CH), jnp.float32),
        pltpu.VMEM((_R_PER_W, _CH), jnp.float32),
        pltpu.SemaphoreType.DMA,
        pltpu.SemaphoreType.DMA,
    ],
)
def _noise_lookup(t_hbm, beta_hbm, alpha_hbm, bt_hbm, at_hbm,
                  idx_v, ob_v, oa_v, sem_g, sem_o):
    wid = lax.axis_index("s") * _NC + lax.axis_index("c")
    row0 = wid * _R_PER_W
    pltpu.sync_copy(t_hbm.at[pl.ds(row0, _R_PER_W)], idx_v)
    copies = []
    for j in range(_R_PER_W):
        copies.append(pltpu.async_copy(beta_hbm.at[idx_v.at[j]], ob_v.at[j], sem_g))
        copies.append(pltpu.async_copy(alpha_hbm.at[idx_v.at[j]], oa_v.at[j], sem_g))
    for c in copies:
        c.wait()
    ob_c = pltpu.async_copy(ob_v, bt_hbm.at[pl.ds(row0, _R_PER_W)], sem_o)
    oa_c = pltpu.async_copy(oa_v, at_hbm.at[pl.ds(row0, _R_PER_W)], sem_o)
    ob_c.wait()
    oa_c.wait()


def kernel(t, beta, alpha):
    bt, at = _noise_lookup(t.reshape(_NROWS, _CH), beta, alpha)
    return bt.reshape(BATCH), at.reshape(BATCH)


# speedup vs baseline: 9.0557x; 1.4073x over previous
import functools

import jax
import jax.numpy as jnp
from jax import lax
from jax.experimental import pallas as pl
from jax.experimental.pallas import tpu as pltpu
from jax.experimental.pallas import tpu_sc as plsc

N_STEPS = 1000
BATCH = 16384

_info = plsc.get_sparse_core_info()
_NS = _info.num_subcores
_CH = 128
_NROWS = BATCH // _CH
_R_PER_W = _NROWS // _NS

_mesh = plsc.VectorSubcoreMesh(core_axis_name="c", subcore_axis_name="s",
                               num_cores=1)


@functools.partial(
    pl.kernel,
    mesh=_mesh,
    out_type=(
        jax.ShapeDtypeStruct((_NROWS, _CH), jnp.float32),
        jax.ShapeDtypeStruct((_NROWS, _CH), jnp.float32),
    ),
    scratch_types=[
        pltpu.VMEM((_R_PER_W, _CH), jnp.int32),
        pltpu.VMEM((_R_PER_W, _CH), jnp.float32),
        pltpu.VMEM((_R_PER_W, _CH), jnp.float32),
        pltpu.VMEM_SHARED((N_STEPS,), jnp.float32),
        pltpu.VMEM_SHARED((N_STEPS,), jnp.float32),
        pltpu.SemaphoreType.DMA,
        pltpu.SemaphoreType.DMA,
    ],
)
def _noise_lookup(t_hbm, beta_hbm, alpha_hbm, bt_hbm, at_hbm,
                  idx_v, ob_v, oa_v, beta_s, alpha_s, sem_g, sem_o):
    sid = lax.axis_index("s")
    row0 = sid * _R_PER_W

    @pl.when(sid == 0)
    def _stage_tables():
        pltpu.sync_copy(beta_hbm, beta_s)
        pltpu.sync_copy(alpha_hbm, alpha_s)

    pltpu.sync_copy(t_hbm.at[pl.ds(row0, _R_PER_W)], idx_v)
    plsc.subcore_barrier()
    copies = []
    for j in range(_R_PER_W):
        copies.append(pltpu.async_copy(beta_s.at[idx_v.at[j]], ob_v.at[j], sem_g))
        copies.append(pltpu.async_copy(alpha_s.at[idx_v.at[j]], oa_v.at[j], sem_g))
    for c in copies:
        c.wait()
    ob_c = pltpu.async_copy(ob_v, bt_hbm.at[pl.ds(row0, _R_PER_W)], sem_o)
    oa_c = pltpu.async_copy(oa_v, at_hbm.at[pl.ds(row0, _R_PER_W)], sem_o)
    ob_c.wait()
    oa_c.wait()


def kernel(t, beta, alpha):
    bt, at = _noise_lookup(t.reshape(_NROWS, _CH), beta, alpha)
    return bt.reshape(BATCH), at.reshape(BATCH)


# speedup vs baseline: 9.0848x; 1.0032x over previous
import functools

import jax
import jax.numpy as jnp
from jax import lax
from jax.experimental import pallas as pl
from jax.experimental.pallas import tpu as pltpu
from jax.experimental.pallas import tpu_sc as plsc

N_STEPS = 1000
BATCH = 16384

_info = plsc.get_sparse_core_info()
_NS = _info.num_subcores
_B_PER_W = BATCH // _NS

_mesh = plsc.VectorSubcoreMesh(core_axis_name="c", subcore_axis_name="s",
                               num_cores=1)


@functools.partial(
    pl.kernel,
    mesh=_mesh,
    out_type=(
        jax.ShapeDtypeStruct((BATCH,), jnp.float32),
        jax.ShapeDtypeStruct((BATCH,), jnp.float32),
    ),
    scratch_types=[
        pltpu.VMEM((_B_PER_W,), jnp.int32),
        pltpu.VMEM((_B_PER_W,), jnp.float32),
        pltpu.VMEM((_B_PER_W,), jnp.float32),
        pltpu.VMEM_SHARED((N_STEPS,), jnp.float32),
        pltpu.VMEM_SHARED((N_STEPS,), jnp.float32),
        pltpu.SemaphoreType.DMA,
        pltpu.SemaphoreType.DMA,
    ],
)
def _noise_lookup(t_hbm, beta_hbm, alpha_hbm, bt_hbm, at_hbm,
                  idx_v, ob_v, oa_v, beta_s, alpha_s, sem_g, sem_o):
    sid = lax.axis_index("s")
    base = sid * _B_PER_W

    @pl.when(sid == 0)
    def _stage_tables():
        pltpu.sync_copy(beta_hbm, beta_s)
        pltpu.sync_copy(alpha_hbm, alpha_s)

    pltpu.sync_copy(t_hbm.at[pl.ds(base, _B_PER_W)], idx_v)
    plsc.subcore_barrier()
    cb = pltpu.async_copy(beta_s.at[idx_v], ob_v, sem_g)
    ca = pltpu.async_copy(alpha_s.at[idx_v], oa_v, sem_g)
    cb.wait()
    ca.wait()
    ob_c = pltpu.async_copy(ob_v, bt_hbm.at[pl.ds(base, _B_PER_W)], sem_o)
    oa_c = pltpu.async_copy(oa_v, at_hbm.at[pl.ds(base, _B_PER_W)], sem_o)
    ob_c.wait()
    oa_c.wait()


def kernel(t, beta, alpha):
    return _noise_lookup(t, beta, alpha)


# speedup vs baseline: 9.4162x; 1.0365x over previous
import functools

import jax
import jax.numpy as jnp
from jax import lax
from jax.experimental import pallas as pl
from jax.experimental.pallas import tpu as pltpu
from jax.experimental.pallas import tpu_sc as plsc

N_STEPS = 1000
BATCH = 16384

_info = plsc.get_sparse_core_info()
_NS = _info.num_subcores
_B_PER_W = BATCH // _NS

_mesh = plsc.VectorSubcoreMesh(core_axis_name="c", subcore_axis_name="s",
                               num_cores=1)


@functools.partial(
    pl.kernel,
    mesh=_mesh,
    out_type=(
        jax.ShapeDtypeStruct((BATCH,), jnp.float32),
        jax.ShapeDtypeStruct((BATCH,), jnp.float32),
    ),
    scratch_types=[
        pltpu.VMEM((_B_PER_W,), jnp.int32),
        pltpu.VMEM((_B_PER_W,), jnp.float32),
        pltpu.VMEM((_B_PER_W,), jnp.float32),
        pltpu.VMEM_SHARED((N_STEPS,), jnp.float32),
        pltpu.VMEM_SHARED((N_STEPS,), jnp.float32),
        pltpu.SemaphoreType.DMA,
        pltpu.SemaphoreType.DMA,
        pltpu.SemaphoreType.DMA,
    ],
)
def _noise_lookup(t_hbm, beta_hbm, alpha_hbm, bt_hbm, at_hbm,
                  idx_v, ob_v, oa_v, beta_s, alpha_s, sem_i, sem_g, sem_o):
    sid = lax.axis_index("s")
    base = sid * _B_PER_W
    idx_c = pltpu.async_copy(t_hbm.at[pl.ds(base, _B_PER_W)], idx_v, sem_i)
    b_c = pltpu.async_copy(beta_hbm, beta_s, sem_i)
    a_c = pltpu.async_copy(alpha_hbm, alpha_s, sem_i)
    idx_c.wait()
    b_c.wait()
    a_c.wait()
    cb = pltpu.async_copy(beta_s.at[idx_v], ob_v, sem_g)
    ca = pltpu.async_copy(alpha_s.at[idx_v], oa_v, sem_g)
    cb.wait()
    ob_c = pltpu.async_copy(ob_v, bt_hbm.at[pl.ds(base, _B_PER_W)], sem_o)
    ca.wait()
    oa_c = pltpu.async_copy(oa_v, at_hbm.at[pl.ds(base, _B_PER_W)], sem_o)
    ob_c.wait()
    oa_c.wait()


def kernel(t, beta, alpha):
    return _noise_lookup(t, beta, alpha)
